# baseline (speedup 1.0000x reference)
import functools

import jax
import jax.numpy as jnp
from jax import lax
from jax.experimental import pallas as pl
from jax.experimental.pallas import tpu as pltpu
from jax.experimental.pallas import tpu_sc as plsc

F32 = jnp.float32
_N = 2048
_E = 32768
_A = 1000
_LOCK = _N + _E
_FEAT = 4 * _N + _LOCK

_HI = lax.Precision.HIGHEST

_NC = 2
_NS = 16
_NW = _NC * _NS
_EPW = _E // _NW
_CH = 128
_NCH = _EPW // _CH

_TW = 64
_HW = 16
_PK = 8
_ZW = _PK * _TW
_HPW = _PK * _HW
_NR = _N // _PK


def _tables_body(x_ref, w_ref, t_ref):
    x = x_ref[...]
    wa = w_ref[:_N, :]
    wb = w_ref[_N:, :]
    t_ref[:, :_TW] = jnp.dot(x, wa - wb, preferred_element_type=F32,
                             precision=_HI)
    t_ref[:, _TW:] = jnp.dot(x, wb, preferred_element_type=F32, precision=_HI)


def _edge_tables(x, w1p):
    mb = 256
    return pl.pallas_call(
        _tables_body,
        grid=(_N // mb,),
        in_specs=[
            pl.BlockSpec((mb, _N), lambda i: (i, 0)),
            pl.BlockSpec((2 * _N, _TW), lambda i: (0, 0)),
        ],
        out_specs=pl.BlockSpec((mb, 2 * _TW), lambda i: (i, 0)),
        out_shape=jax.ShapeDtypeStruct((_N, 2 * _TW), F32),
    )(x, w1p)


def _gather_body(tab, didx, sidx, out, shared, idx_v, bufd, bufs, zbuf,
                 semd, sems):
    wid = lax.axis_index("s") * _NC + lax.axis_index("c")
    sid = lax.axis_index("s")

    @pl.when(sid == 0)
    def _():
        pltpu.sync_copy(tab, shared)

    pltpu.sync_copy(didx.at[pl.ds(wid * _NCH, _NCH)], idx_v.at[0])
    pltpu.sync_copy(sidx.at[pl.ds(wid * _NCH, _NCH)], idx_v.at[1])
    plsc.subcore_barrier()

    for r in range(_NCH):
        cd = pltpu.async_copy(shared.at[idx_v.at[0, r]], bufd, semd)
        cs = pltpu.async_copy(shared.at[idx_v.at[1, r]], bufs, sems)
        cd.wait()
        cs.wait()

        def compute(e, carry):
            er = e >> 3
            ec = (e & 7) * _TW
            for c in range(4):
                a = bufd[e, pl.ds(c * 16, 16)]
                b = bufs[e, pl.ds(_TW + c * 16, 16)]
                zbuf[er, pl.ds(ec + c * 16, 16)] = a + b
            return carry

        lax.fori_loop(0, _CH, compute, 0)
        pltpu.sync_copy(
            zbuf, out.at[pl.ds((wid * _NCH + r) * (_CH // _PK), _CH // _PK)]
        )


_gather_rows = functools.partial(
    pl.kernel,
    out_type=jax.ShapeDtypeStruct((_E // _PK, _ZW), F32),
    mesh=plsc.VectorSubcoreMesh(core_axis_name="c", subcore_axis_name="s"),
    scratch_types=[
        pltpu.VMEM_SHARED((_N, 2 * _TW), F32),
        pltpu.VMEM((2, _NCH, _CH), jnp.int32),
        pltpu.VMEM((_CH, 2 * _TW), F32),
        pltpu.VMEM((_CH, 2 * _TW), F32),
        pltpu.VMEM((_CH // _PK, _ZW), F32),
        pltpu.SemaphoreType.DMA,
        pltpu.SemaphoreType.DMA,
    ],
)(_gather_body)


def _mlp_body(z_ref, b1_ref, w2_ref, b2_ref, w3_ref, b3_ref, h_ref):
    z1 = z_ref[...] + b1_ref[...]
    z1 = z1 * jax.nn.sigmoid(z1)
    z2 = jnp.dot(z1, w2_ref[...], preferred_element_type=F32, precision=_HI)
    z2 = z2 + b2_ref[...]
    z2 = z2 * jax.nn.sigmoid(z2)
    z3 = jnp.dot(z2, w3_ref[...], preferred_element_type=F32, precision=_HI)
    z3 = z3 + b3_ref[...]
    h_ref[...] = z3 * jax.nn.sigmoid(z3)


def _edge_mlp(rows_z, b1t, bd2, b2t, bd3, b3t):
    be = 512
    const = lambda i: (0, 0)
    return pl.pallas_call(
        _mlp_body,
        grid=(_E // _PK // be,),
        in_specs=[
            pl.BlockSpec((be, _ZW), lambda i: (i, 0)),
            pl.BlockSpec((1, _ZW), const),
            pl.BlockSpec((_ZW, _HPW), const),
            pl.BlockSpec((1, _HPW), const),
            pl.BlockSpec((_HPW, _HPW), const),
            pl.BlockSpec((1, _HPW), const),
        ],
        out_specs=pl.BlockSpec((be, _HPW), lambda i: (i, 0)),
        out_shape=jax.ShapeDtypeStruct((_E // _PK, _HPW), F32),
    )(rows_z, b1t, bd2, b2t, bd3, b3t)


def _segmax_body(h_hbm, d_hbm, init_hbm, parts, hbuf, dbuf, acc):
    wid = lax.axis_index("s") * _NC + lax.axis_index("c")
    rpw = _EPW // _PK
    pltpu.sync_copy(init_hbm, acc)
    pltpu.sync_copy(h_hbm.at[pl.ds(wid * rpw, rpw)], hbuf)
    pltpu.sync_copy(d_hbm.at[pl.ds(wid * _NCH, _NCH)], dbuf)

    def body(g, carry):
        row = g >> 3
        col = (g & 7) * 16
        dvec = dbuf[row, pl.ds(col, 16)]
        for j in range(16):
            d = dvec[j]
            dr = d >> 3
            dc = (d & 7) * _HW
            hv = hbuf[2 * g + (j >> 3), pl.ds((j & 7) * _HW, _HW)]
            cur = acc[dr, pl.ds(dc, _HW)]
            acc[dr, pl.ds(dc, _HW)] = jnp.maximum(cur, hv)
        return carry

    lax.fori_loop(0, _EPW // 16, body, 0)
    pltpu.sync_copy(acc, parts.at[wid])


_segmax = functools.partial(
    pl.kernel,
    out_type=jax.ShapeDtypeStruct((_NW, _NR, _HPW), F32),
    mesh=plsc.VectorSubcoreMesh(core_axis_name="c", subcore_axis_name="s"),
    scratch_types=[
        pltpu.VMEM((_EPW // _PK, _HPW), F32),
        pltpu.VMEM((_NCH, _CH), jnp.int32),
        pltpu.VMEM((_NR, _HPW), F32),
    ],
)(_segmax_body)


def _maxred_body(p_ref, q_ref):
    m = p_ref[0]
    for i in range(1, _NW):
        m = jnp.maximum(m, p_ref[i])
    m = jnp.where(jnp.isfinite(m), m, 0.0)
    q_ref[...] = jnp.concatenate(
        [m[:, j * _HW:j * _HW + 4] for j in range(_PK)], axis=1
    )


def _max_reduce(parts):
    return pl.pallas_call(
        _maxred_body,
        in_specs=[pl.BlockSpec((_NW, _NR, _HPW), lambda: (0, 0, 0))],
        out_specs=pl.BlockSpec((_NR, 4 * _PK), lambda: (0, 0)),
        out_shape=jax.ShapeDtypeStruct((_NR, 4 * _PK), F32),
    )(parts)


_KB = 2048
_NKB = _FEAT // _KB


_NKQ = (4 * _N) // _KB
_NKL = _LOCK // _KB


def _heads_locked_body(f_ref, wp_ref, acc_ref, accp):
    k = pl.program_id(0)
    pp = jnp.dot(f_ref[...], wp_ref[...], preferred_element_type=F32)

    @pl.when(k == 0)
    def _():
        accp[...] = pp

    @pl.when(k > 0)
    def _():
        accp[...] += pp

    @pl.when(k == _NKL - 1)
    def _():
        acc_ref[...] = accp[...]


def _heads_locked(locked2d, wp):
    const = lambda k: (0, 0)
    return pl.pallas_call(
        _heads_locked_body,
        grid=(_NKL,),
        in_specs=[
            pl.BlockSpec((1, _KB), lambda k: (0, k)),
            pl.BlockSpec((_KB, _A), lambda k: (k + _NKQ, 0)),
        ],
        out_specs=pl.BlockSpec((1, _A), const),
        out_shape=jax.ShapeDtypeStruct((1, _A), F32),
        scratch_shapes=[
            pltpu.VMEM((1, _A), F32),
        ],
    )(locked2d, wp)


def _heads_q_body(f_ref, wp_ref, part_ref, bp_ref, pol_ref, accp):
    k = pl.program_id(0)
    pp = jnp.dot(f_ref[...], wp_ref[...], preferred_element_type=F32)

    @pl.when(k == 0)
    def _():
        accp[...] = pp

    @pl.when(k > 0)
    def _():
        accp[...] += pp

    @pl.when(k == _NKQ - 1)
    def _():
        logits = accp[...] + part_ref[...] + bp_ref[...]
        m = jnp.max(logits, axis=-1, keepdims=True)
        ez = jnp.exp(logits - m)
        pol_ref[...] = ez / jnp.sum(ez, axis=-1, keepdims=True)


def _heads_q(qfeat, wp, part, bp):
    const = lambda k: (0, 0)
    return pl.pallas_call(
        _heads_q_body,
        grid=(_NKQ,),
        in_specs=[
            pl.BlockSpec((1, _KB), lambda k: (0, k)),
            pl.BlockSpec((_KB, _A), lambda k: (k, 0)),
            pl.BlockSpec((1, _A), const),
            pl.BlockSpec((1, _A), const),
        ],
        out_specs=pl.BlockSpec((1, _A), const),
        out_shape=jax.ShapeDtypeStruct((1, _A), F32),
        scratch_shapes=[
            pltpu.VMEM((1, _A), F32),
        ],
    )(qfeat, wp, part, bp)


def kernel(qubit_interactions, locked_edges, edges, W1, b1, W2, b2, W3, b3,
           Wv1, bv1, Wv2, bv2, Wv3, bv3, Wp, bp):
    x = qubit_interactions
    w1p = jnp.zeros((2 * _N, _TW), F32).at[:, :50].set(W1)
    tab = _edge_tables(x, w1p)

    dst2d = edges[1].reshape(_E // _CH, _CH)
    src2d = edges[0].reshape(_E // _CH, _CH)
    rows_z = _gather_rows(tab, dst2d, src2d)

    b1p = jnp.zeros((1, _TW), F32).at[0, :50].set(b1)
    w2p = jnp.zeros((_TW, _HW), F32).at[:50, :10].set(W2)
    b2p = jnp.zeros((1, _HW), F32).at[0, :10].set(b2)
    w3p = jnp.zeros((_HW, _HW), F32).at[:10, :4].set(W3)
    b3p = jnp.zeros((1, _HW), F32).at[0, :4].set(b3)
    eye8 = jnp.eye(_PK, dtype=F32)
    b1t = jnp.tile(b1p, (1, _PK))
    bd2 = jnp.kron(eye8, w2p)
    b2t = jnp.tile(b2p, (1, _PK))
    bd3 = jnp.kron(eye8, w3p)
    b3t = jnp.tile(b3p, (1, _PK))
    h = _edge_mlp(rows_z, b1t, bd2, b2t, bd3, b3t)

    ninit = jnp.full((_NR, _HPW), -jnp.inf, F32)
    parts = _segmax(h, dst2d, ninit)
    q2d = _max_reduce(parts)

    part = _heads_locked(locked_edges.reshape(1, _LOCK), Wp)
    q = q2d.reshape(-1)
    pol = _heads_q(q.reshape(1, 4 * _N), Wp, part, bp.reshape(1, _A))
    feat = jnp.concatenate([q, locked_edges])
    hv = feat @ Wv1 + bv1
    hv = hv * jax.nn.sigmoid(hv)
    hv = hv @ Wv2 + bv2
    hv = hv * jax.nn.sigmoid(hv)
    val = hv @ Wv3 + bv3
    return (val, pol.reshape(_A))


# speedup vs baseline: 1.0088x; 1.0088x over previous
import functools

import jax
import jax.numpy as jnp
from jax import lax
from jax.experimental import pallas as pl
from jax.experimental.pallas import tpu as pltpu
from jax.experimental.pallas import tpu_sc as plsc

F32 = jnp.float32
_N = 2048
_E = 32768
_A = 1000
_LOCK = _N + _E
_FEAT = 4 * _N + _LOCK

_HI = lax.Precision.HIGHEST

_NC = 2
_NS = 16
_NW = _NC * _NS
_EPW = _E // _NW
_CH = 128
_NCH = _EPW // _CH

_TW = 64
_HW = 16
_PK = 8
_ZW = _PK * _TW
_HPW = _PK * _HW
_NR = _N // _PK


def _tables_body(x_ref, w_ref, t_ref):
    x = x_ref[...]
    wa = w_ref[:_N, :]
    wb = w_ref[_N:, :]
    t_ref[:, :_TW] = jnp.dot(x, wa - wb, preferred_element_type=F32,
                             precision=_HI)
    t_ref[:, _TW:] = jnp.dot(x, wb, preferred_element_type=F32, precision=_HI)


def _edge_tables(x, w1p):
    mb = 256
    return pl.pallas_call(
        _tables_body,
        grid=(_N // mb,),
        in_specs=[
            pl.BlockSpec((mb, _N), lambda i: (i, 0)),
            pl.BlockSpec((2 * _N, _TW), lambda i: (0, 0)),
        ],
        out_specs=pl.BlockSpec((mb, 2 * _TW), lambda i: (i, 0)),
        out_shape=jax.ShapeDtypeStruct((_N, 2 * _TW), F32),
    )(x, w1p)


def _gather_body(tab, didx, sidx, out, shared, idx_v, bufd, bufs, zbuf,
                 semd, sems):
    wid = lax.axis_index("s") * _NC + lax.axis_index("c")
    sid = lax.axis_index("s")

    @pl.when(sid == 0)
    def _():
        pltpu.sync_copy(tab, shared)

    pltpu.sync_copy(didx.at[pl.ds(wid * _NCH, _NCH)], idx_v.at[0])
    pltpu.sync_copy(sidx.at[pl.ds(wid * _NCH, _NCH)], idx_v.at[1])
    plsc.subcore_barrier()

    for r in range(_NCH):
        cd = pltpu.async_copy(shared.at[idx_v.at[0, r]], bufd, semd)
        cs = pltpu.async_copy(shared.at[idx_v.at[1, r]], bufs, sems)
        cd.wait()
        cs.wait()

        def compute(e, carry):
            er = e >> 3
            ec = (e & 7) * _TW
            for c in range(4):
                a = bufd[e, pl.ds(c * 16, 16)]
                b = bufs[e, pl.ds(_TW + c * 16, 16)]
                zbuf[er, pl.ds(ec + c * 16, 16)] = a + b
            return carry

        lax.fori_loop(0, _CH, compute, 0)
        pltpu.sync_copy(
            zbuf, out.at[pl.ds((wid * _NCH + r) * (_CH // _PK), _CH // _PK)]
        )


_gather_rows = functools.partial(
    pl.kernel,
    out_type=jax.ShapeDtypeStruct((_E // _PK, _ZW), F32),
    mesh=plsc.VectorSubcoreMesh(core_axis_name="c", subcore_axis_name="s"),
    scratch_types=[
        pltpu.VMEM_SHARED((_N, 2 * _TW), F32),
        pltpu.VMEM((2, _NCH, _CH), jnp.int32),
        pltpu.VMEM((_CH, 2 * _TW), F32),
        pltpu.VMEM((_CH, 2 * _TW), F32),
        pltpu.VMEM((_CH // _PK, _ZW), F32),
        pltpu.SemaphoreType.DMA,
        pltpu.SemaphoreType.DMA,
    ],
)(_gather_body)


def _mlp_body(z_ref, b1_ref, w2_ref, b2_ref, w3_ref, b3_ref, h_ref):
    z1 = z_ref[...] + b1_ref[...]
    z1 = z1 * jax.nn.sigmoid(z1)
    z2 = jnp.dot(z1, w2_ref[...], preferred_element_type=F32, precision=_HI)
    z2 = z2 + b2_ref[...]
    z2 = z2 * jax.nn.sigmoid(z2)
    z3 = jnp.dot(z2, w3_ref[...], preferred_element_type=F32, precision=_HI)
    z3 = z3 + b3_ref[...]
    h_ref[...] = z3 * jax.nn.sigmoid(z3)


def _edge_mlp(rows_z, b1t, bd2, b2t, bd3, b3t):
    be = 512
    const = lambda i: (0, 0)
    return pl.pallas_call(
        _mlp_body,
        grid=(_E // _PK // be,),
        in_specs=[
            pl.BlockSpec((be, _ZW), lambda i: (i, 0)),
            pl.BlockSpec((1, _ZW), const),
            pl.BlockSpec((_ZW, _HPW), const),
            pl.BlockSpec((1, _HPW), const),
            pl.BlockSpec((_HPW, _HPW), const),
            pl.BlockSpec((1, _HPW), const),
        ],
        out_specs=pl.BlockSpec((be, _HPW), lambda i: (i, 0)),
        out_shape=jax.ShapeDtypeStruct((_E // _PK, _HPW), F32),
    )(rows_z, b1t, bd2, b2t, bd3, b3t)


def _segmax_body(h_hbm, d_hbm, init_hbm, parts, hbuf, dbuf, acc):
    wid = lax.axis_index("s") * _NC + lax.axis_index("c")
    rpw = _EPW // _PK
    pltpu.sync_copy(init_hbm, acc)
    pltpu.sync_copy(h_hbm.at[pl.ds(wid * rpw, rpw)], hbuf)
    pltpu.sync_copy(d_hbm.at[pl.ds(wid * _NCH, _NCH)], dbuf)

    def body(g, carry):
        row = g >> 3
        col = (g & 7) * 16
        dvec = dbuf[row, pl.ds(col, 16)]
        for j in range(16):
            d = dvec[j]
            dr = d >> 3
            dc = (d & 7) * _HW
            hv = hbuf[2 * g + (j >> 3), pl.ds((j & 7) * _HW, _HW)]
            cur = acc[dr, pl.ds(dc, _HW)]
            acc[dr, pl.ds(dc, _HW)] = jnp.maximum(cur, hv)
        return carry

    lax.fori_loop(0, _EPW // 16, body, 0)
    pltpu.sync_copy(acc, parts.at[wid])


_segmax = functools.partial(
    pl.kernel,
    out_type=jax.ShapeDtypeStruct((_NW, _NR, _HPW), F32),
    mesh=plsc.VectorSubcoreMesh(core_axis_name="c", subcore_axis_name="s"),
    scratch_types=[
        pltpu.VMEM((_EPW // _PK, _HPW), F32),
        pltpu.VMEM((_NCH, _CH), jnp.int32),
        pltpu.VMEM((_NR, _HPW), F32),
    ],
)(_segmax_body)


def _maxred_body(p_ref, q_ref):
    m = p_ref[0]
    for i in range(1, _NW):
        m = jnp.maximum(m, p_ref[i])
    m = jnp.where(jnp.isfinite(m), m, 0.0)
    q_ref[...] = jnp.concatenate(
        [m[:, j * _HW:j * _HW + 4] for j in range(_PK)], axis=1
    )


def _max_reduce(parts):
    return pl.pallas_call(
        _maxred_body,
        in_specs=[pl.BlockSpec((_NW, _NR, _HPW), lambda: (0, 0, 0))],
        out_specs=pl.BlockSpec((_NR, 4 * _PK), lambda: (0, 0)),
        out_shape=jax.ShapeDtypeStruct((_NR, 4 * _PK), F32),
    )(parts)


_KB = 2048
_NKB = _FEAT // _KB


def _heads_body(f_ref, wp_ref, bp_ref, pol_ref, accp):
    k = pl.program_id(0)
    pp = jnp.dot(f_ref[...], wp_ref[...], preferred_element_type=F32)

    @pl.when(k == 0)
    def _():
        accp[...] = pp

    @pl.when(k > 0)
    def _():
        accp[...] += pp

    @pl.when(k == _NKB - 1)
    def _():
        logits = accp[...] + bp_ref[...]
        m = jnp.max(logits, axis=-1, keepdims=True)
        ez = jnp.exp(logits - m)
        pol_ref[...] = ez / jnp.sum(ez, axis=-1, keepdims=True)


def _heads(feat, wp, bp):
    const = lambda k: (0, 0)
    return pl.pallas_call(
        _heads_body,
        grid=(_NKB,),
        in_specs=[
            pl.BlockSpec((1, _KB), lambda k: (0, k)),
            pl.BlockSpec((_KB, _A), lambda k: (k, 0)),
            pl.BlockSpec((1, _A), const),
        ],
        out_specs=pl.BlockSpec((1, _A), const),
        out_shape=jax.ShapeDtypeStruct((1, _A), F32),
        scratch_shapes=[
            pltpu.VMEM((1, _A), F32),
        ],
    )(feat, wp, bp)


def kernel(qubit_interactions, locked_edges, edges, W1, b1, W2, b2, W3, b3,
           Wv1, bv1, Wv2, bv2, Wv3, bv3, Wp, bp):
    x = qubit_interactions
    w1p = jnp.zeros((2 * _N, _TW), F32).at[:, :50].set(W1)
    tab = _edge_tables(x, w1p)

    dst2d = edges[1].reshape(_E // _CH, _CH)
    src2d = edges[0].reshape(_E // _CH, _CH)
    rows_z = _gather_rows(tab, dst2d, src2d)

    b1p = jnp.zeros((1, _TW), F32).at[0, :50].set(b1)
    w2p = jnp.zeros((_TW, _HW), F32).at[:50, :10].set(W2)
    b2p = jnp.zeros((1, _HW), F32).at[0, :10].set(b2)
    w3p = jnp.zeros((_HW, _HW), F32).at[:10, :4].set(W3)
    b3p = jnp.zeros((1, _HW), F32).at[0, :4].set(b3)
    eye8 = jnp.eye(_PK, dtype=F32)
    b1t = jnp.tile(b1p, (1, _PK))
    bd2 = jnp.kron(eye8, w2p)
    b2t = jnp.tile(b2p, (1, _PK))
    bd3 = jnp.kron(eye8, w3p)
    b3t = jnp.tile(b3p, (1, _PK))
    h = _edge_mlp(rows_z, b1t, bd2, b2t, bd3, b3t)

    ninit = jnp.full((_NR, _HPW), -jnp.inf, F32)
    parts = _segmax(h, dst2d, ninit)
    q2d = _max_reduce(parts)

    feat = jnp.concatenate([q2d.reshape(-1), locked_edges])
    pol = _heads(feat.reshape(1, _FEAT), Wp, bp.reshape(1, _A))
    hv = feat @ Wv1 + bv1
    hv = hv * jax.nn.sigmoid(hv)
    hv = hv @ Wv2 + bv2
    hv = hv * jax.nn.sigmoid(hv)
    val = hv @ Wv3 + bv3
    return (val, pol.reshape(_A))


# speedup vs baseline: 1.0388x; 1.0297x over previous
import functools

import jax
import jax.numpy as jnp
from jax import lax
from jax.experimental import pallas as pl
from jax.experimental.pallas import tpu as pltpu
from jax.experimental.pallas import tpu_sc as plsc

F32 = jnp.float32
_N = 2048
_E = 32768
_A = 1000
_LOCK = _N + _E
_FEAT = 4 * _N + _LOCK

_HI = lax.Precision.HIGHEST

_NC = 2
_NS = 16
_NW = _NC * _NS
_EPW = _E // _NW
_CH = 128
_NCH = _EPW // _CH

_TW = 64
_HW = 16
_PK = 8
_ZW = _PK * _TW
_HPW = _PK * _HW
_NR = _N // _PK


def _tables_body(x_ref, w_ref, t_ref):
    x = x_ref[...]
    wa = w_ref[:_N, :]
    wb = w_ref[_N:, :]
    t_ref[:, :_TW] = jnp.dot(x, wa - wb, preferred_element_type=F32,
                             precision=_HI)
    t_ref[:, _TW:] = jnp.dot(x, wb, preferred_element_type=F32, precision=_HI)


def _edge_tables(x, w1p):
    mb = 256
    return pl.pallas_call(
        _tables_body,
        grid=(_N // mb,),
        in_specs=[
            pl.BlockSpec((mb, _N), lambda i: (i, 0)),
            pl.BlockSpec((2 * _N, _TW), lambda i: (0, 0)),
        ],
        out_specs=pl.BlockSpec((mb, 2 * _TW), lambda i: (i, 0)),
        out_shape=jax.ShapeDtypeStruct((_N, 2 * _TW), F32),
    )(x, w1p)


def _gather_body(tab, didx, sidx, out, shared, idx_v, bufd, bufs, zbuf,
                 semd, sems):
    wid = lax.axis_index("s") * _NC + lax.axis_index("c")
    sid = lax.axis_index("s")

    @pl.when(sid == 0)
    def _():
        pltpu.sync_copy(tab, shared)

    pltpu.sync_copy(didx.at[pl.ds(wid * _NCH, _NCH)], idx_v.at[0])
    pltpu.sync_copy(sidx.at[pl.ds(wid * _NCH, _NCH)], idx_v.at[1])
    plsc.subcore_barrier()

    def start(r, p):
        return (
            pltpu.async_copy(shared.at[idx_v.at[0, r]], bufd.at[p], semd),
            pltpu.async_copy(shared.at[idx_v.at[1, r]], bufs.at[p], sems),
        )

    cps = start(0, 0)
    for r in range(_NCH):
        p = r & 1
        cps[0].wait()
        cps[1].wait()
        if r + 1 < _NCH:
            cps = start(r + 1, 1 - p)

        def compute(e, carry):
            er = e >> 3
            ec = (e & 7) * _TW
            for c in range(4):
                a = bufd[p, e, pl.ds(c * 16, 16)]
                b = bufs[p, e, pl.ds(_TW + c * 16, 16)]
                zbuf[er, pl.ds(ec + c * 16, 16)] = a + b
            return carry

        lax.fori_loop(0, _CH, compute, 0)
        pltpu.sync_copy(
            zbuf, out.at[pl.ds((wid * _NCH + r) * (_CH // _PK), _CH // _PK)]
        )


_gather_rows = functools.partial(
    pl.kernel,
    out_type=jax.ShapeDtypeStruct((_E // _PK, _ZW), F32),
    mesh=plsc.VectorSubcoreMesh(core_axis_name="c", subcore_axis_name="s"),
    scratch_types=[
        pltpu.VMEM_SHARED((_N, 2 * _TW), F32),
        pltpu.VMEM((2, _NCH, _CH), jnp.int32),
        pltpu.VMEM((2, _CH, 2 * _TW), F32),
        pltpu.VMEM((2, _CH, 2 * _TW), F32),
        pltpu.VMEM((_CH // _PK, _ZW), F32),
        pltpu.SemaphoreType.DMA,
        pltpu.SemaphoreType.DMA,
    ],
)(_gather_body)


def _mlp_body(z_ref, b1_ref, w2_ref, b2_ref, w3_ref, b3_ref, h_ref):
    z1 = z_ref[...] + b1_ref[...]
    z1 = z1 * jax.nn.sigmoid(z1)
    z2 = jnp.dot(z1, w2_ref[...], preferred_element_type=F32, precision=_HI)
    z2 = z2 + b2_ref[...]
    z2 = z2 * jax.nn.sigmoid(z2)
    z3 = jnp.dot(z2, w3_ref[...], preferred_element_type=F32, precision=_HI)
    z3 = z3 + b3_ref[...]
    h_ref[...] = z3 * jax.nn.sigmoid(z3)


def _edge_mlp(rows_z, b1t, bd2, b2t, bd3, b3t):
    be = 512
    const = lambda i: (0, 0)
    return pl.pallas_call(
        _mlp_body,
        grid=(_E // _PK // be,),
        in_specs=[
            pl.BlockSpec((be, _ZW), lambda i: (i, 0)),
            pl.BlockSpec((1, _ZW), const),
            pl.BlockSpec((_ZW, _HPW), const),
            pl.BlockSpec((1, _HPW), const),
            pl.BlockSpec((_HPW, _HPW), const),
            pl.BlockSpec((1, _HPW), const),
        ],
        out_specs=pl.BlockSpec((be, _HPW), lambda i: (i, 0)),
        out_shape=jax.ShapeDtypeStruct((_E // _PK, _HPW), F32),
    )(rows_z, b1t, bd2, b2t, bd3, b3t)


def _segmax_body(h_hbm, d_hbm, init_hbm, parts, hbuf, dbuf, acc):
    wid = lax.axis_index("s") * _NC + lax.axis_index("c")
    rpw = _EPW // _PK
    pltpu.sync_copy(init_hbm, acc)
    pltpu.sync_copy(h_hbm.at[pl.ds(wid * rpw, rpw)], hbuf)
    pltpu.sync_copy(d_hbm.at[pl.ds(wid * _NCH, _NCH)], dbuf)

    def body(g, carry):
        row = g >> 3
        col = (g & 7) * 16
        dvec = dbuf[row, pl.ds(col, 16)]
        for j in range(16):
            d = dvec[j]
            dr = d >> 3
            dc = (d & 7) * _HW
            hv = hbuf[2 * g + (j >> 3), pl.ds((j & 7) * _HW, _HW)]
            cur = acc[dr, pl.ds(dc, _HW)]
            acc[dr, pl.ds(dc, _HW)] = jnp.maximum(cur, hv)
        return carry

    lax.fori_loop(0, _EPW // 16, body, 0)
    pltpu.sync_copy(acc, parts.at[wid])


_segmax = functools.partial(
    pl.kernel,
    out_type=jax.ShapeDtypeStruct((_NW, _NR, _HPW), F32),
    mesh=plsc.VectorSubcoreMesh(core_axis_name="c", subcore_axis_name="s"),
    scratch_types=[
        pltpu.VMEM((_EPW // _PK, _HPW), F32),
        pltpu.VMEM((_NCH, _CH), jnp.int32),
        pltpu.VMEM((_NR, _HPW), F32),
    ],
)(_segmax_body)


def _maxred_body(p_ref, q_ref):
    m = p_ref[0]
    for i in range(1, _NW):
        m = jnp.maximum(m, p_ref[i])
    m = jnp.where(jnp.isfinite(m), m, 0.0)
    q_ref[...] = jnp.concatenate(
        [m[:, j * _HW:j * _HW + 4] for j in range(_PK)], axis=1
    )


def _max_reduce(parts):
    return pl.pallas_call(
        _maxred_body,
        in_specs=[pl.BlockSpec((_NW, _NR, _HPW), lambda: (0, 0, 0))],
        out_specs=pl.BlockSpec((_NR, 4 * _PK), lambda: (0, 0)),
        out_shape=jax.ShapeDtypeStruct((_NR, 4 * _PK), F32),
    )(parts)


_KB = 2048
_NKB = _FEAT // _KB


def _heads_body(f_ref, wp_ref, bp_ref, pol_ref, accp):
    k = pl.program_id(0)
    pp = jnp.dot(f_ref[...], wp_ref[...], preferred_element_type=F32)

    @pl.when(k == 0)
    def _():
        accp[...] = pp

    @pl.when(k > 0)
    def _():
        accp[...] += pp

    @pl.when(k == _NKB - 1)
    def _():
        logits = accp[...] + bp_ref[...]
        m = jnp.max(logits, axis=-1, keepdims=True)
        ez = jnp.exp(logits - m)
        pol_ref[...] = ez / jnp.sum(ez, axis=-1, keepdims=True)


def _heads(feat, wp, bp):
    const = lambda k: (0, 0)
    return pl.pallas_call(
        _heads_body,
        grid=(_NKB,),
        in_specs=[
            pl.BlockSpec((1, _KB), lambda k: (0, k)),
            pl.BlockSpec((_KB, _A), lambda k: (k, 0)),
            pl.BlockSpec((1, _A), const),
        ],
        out_specs=pl.BlockSpec((1, _A), const),
        out_shape=jax.ShapeDtypeStruct((1, _A), F32),
        scratch_shapes=[
            pltpu.VMEM((1, _A), F32),
        ],
    )(feat, wp, bp)


def kernel(qubit_interactions, locked_edges, edges, W1, b1, W2, b2, W3, b3,
           Wv1, bv1, Wv2, bv2, Wv3, bv3, Wp, bp):
    x = qubit_interactions
    w1p = jnp.zeros((2 * _N, _TW), F32).at[:, :50].set(W1)
    tab = _edge_tables(x, w1p)

    dst2d = edges[1].reshape(_E // _CH, _CH)
    src2d = edges[0].reshape(_E // _CH, _CH)
    rows_z = _gather_rows(tab, dst2d, src2d)

    b1p = jnp.zeros((1, _TW), F32).at[0, :50].set(b1)
    w2p = jnp.zeros((_TW, _HW), F32).at[:50, :10].set(W2)
    b2p = jnp.zeros((1, _HW), F32).at[0, :10].set(b2)
    w3p = jnp.zeros((_HW, _HW), F32).at[:10, :4].set(W3)
    b3p = jnp.zeros((1, _HW), F32).at[0, :4].set(b3)
    eye8 = jnp.eye(_PK, dtype=F32)
    b1t = jnp.tile(b1p, (1, _PK))
    bd2 = jnp.kron(eye8, w2p)
    b2t = jnp.tile(b2p, (1, _PK))
    bd3 = jnp.kron(eye8, w3p)
    b3t = jnp.tile(b3p, (1, _PK))
    h = _edge_mlp(rows_z, b1t, bd2, b2t, bd3, b3t)

    ninit = jnp.full((_NR, _HPW), -jnp.inf, F32)
    parts = _segmax(h, dst2d, ninit)
    q2d = _max_reduce(parts)

    feat = jnp.concatenate([q2d.reshape(-1), locked_edges])
    pol = _heads(feat.reshape(1, _FEAT), Wp, bp.reshape(1, _A))
    hv = feat @ Wv1 + bv1
    hv = hv * jax.nn.sigmoid(hv)
    hv = hv @ Wv2 + bv2
    hv = hv * jax.nn.sigmoid(hv)
    val = hv @ Wv3 + bv3
    return (val, pol.reshape(_A))
